# initial kernel scaffold (unmeasured)
import jax
import jax.numpy as jnp
from jax import lax
from jax.experimental import pallas as pl
from jax.experimental.pallas import tpu as pltpu

N_DEV = 16


def kernel(partial, resid, gamma):
    _, M, D = partial.shape
    CH = M // N_DEV
    p2 = partial.reshape(M, D)
    g2 = gamma.reshape(1, D)

    def body(p_ref, r_ref, g_ref, out_ref,
             rs_send, rs_recv, my_norm, ag_recv,
             rs_send_sems, rs_recv_sems, ag_send_sems, ag_recv_sems):
        my = lax.axis_index("i")
        right = lax.rem(my + 1, N_DEV)

        for s in range(N_DEV - 1):
            send_c = lax.rem(my - s + N_DEV, N_DEV)
            chunk = p_ref[pl.ds(send_c * CH, CH), :]
            if s > 0:
                chunk = chunk + rs_recv[s - 1].astype(jnp.float32)
            rs_send[s] = chunk.astype(jnp.bfloat16)
            rdma = pltpu.make_async_remote_copy(
                src_ref=rs_send.at[s],
                dst_ref=rs_recv.at[s],
                send_sem=rs_send_sems.at[s],
                recv_sem=rs_recv_sems.at[s],
                device_id=(right,),
                device_id_type=pl.DeviceIdType.MESH,
            )
            rdma.start()
            rdma.wait()

        own = lax.rem(my + 1, N_DEV)
        osl = pl.ds(own * CH, CH)
        y = (p_ref[osl, :] + rs_recv[N_DEV - 2].astype(jnp.float32)
             + r_ref[osl, :])
        ms = jnp.mean(y * y, axis=-1, keepdims=True)
        o = y * lax.rsqrt(ms + 1e-6) * g_ref[...]
        out_ref[osl, :] = o
        my_norm[...] = o.astype(jnp.bfloat16)

        for s in range(N_DEV - 1):
            src = my_norm if s == 0 else ag_recv.at[s - 1]
            rdma = pltpu.make_async_remote_copy(
                src_ref=src,
                dst_ref=ag_recv.at[s],
                send_sem=ag_send_sems.at[s],
                recv_sem=ag_recv_sems.at[s],
                device_id=(right,),
                device_id_type=pl.DeviceIdType.MESH,
            )
            rdma.start()
            rdma.wait()
            idx = lax.rem(my - s + N_DEV, N_DEV)
            out_ref[pl.ds(idx * CH, CH), :] = ag_recv[s].astype(jnp.float32)

    return pl.pallas_call(
        body,
        out_shape=jax.ShapeDtypeStruct((M, D), jnp.float32),
        in_specs=[pl.BlockSpec(memory_space=pltpu.VMEM)] * 3,
        out_specs=pl.BlockSpec(memory_space=pltpu.VMEM),
        scratch_shapes=[
            pltpu.VMEM((N_DEV - 1, CH, D), jnp.bfloat16),
            pltpu.VMEM((N_DEV - 1, CH, D), jnp.bfloat16),
            pltpu.VMEM((CH, D), jnp.bfloat16),
            pltpu.VMEM((N_DEV - 1, CH, D), jnp.bfloat16),
            pltpu.SemaphoreType.DMA((N_DEV - 1,)),
            pltpu.SemaphoreType.DMA((N_DEV - 1,)),
            pltpu.SemaphoreType.DMA((N_DEV - 1,)),
            pltpu.SemaphoreType.DMA((N_DEV - 1,)),
        ],
    )(p2, resid, g2)


# baseline (device time: 260039 ns/iter reference)
import jax
import jax.numpy as jnp
from jax import lax
from jax.experimental import pallas as pl
from jax.experimental.pallas import tpu as pltpu

N_DEV = 16


def kernel(partial, resid, gamma):
    _, M, D = partial.shape
    CH = M // N_DEV
    p2 = partial.reshape(M, D)
    g2 = gamma.reshape(1, D)

    def body(p_ref, r_ref, g_ref, out_ref,
             rs_send, rs_recv, my_norm, ag_recv, r_chunk,
             rs_send_sems, rs_recv_sems, ag_send_sems, ag_recv_sems,
             local_sem):
        my = lax.axis_index("i")
        right = lax.rem(my + 1, N_DEV)
        own = lax.rem(my + 1, N_DEV)
        osl = pl.ds(own * CH, CH)

        r_copy = pltpu.make_async_copy(r_ref.at[osl, :], r_chunk, local_sem)
        r_copy.start()

        for s in range(N_DEV - 1):
            send_c = lax.rem(my - s + N_DEV, N_DEV)
            chunk = p_ref[pl.ds(send_c * CH, CH), :]
            if s > 0:
                chunk = chunk + rs_recv[s - 1].astype(jnp.float32)
            rs_send[s % 2] = chunk.astype(jnp.bfloat16)
            rdma = pltpu.make_async_remote_copy(
                src_ref=rs_send.at[s % 2],
                dst_ref=rs_recv.at[s],
                send_sem=rs_send_sems.at[s],
                recv_sem=rs_recv_sems.at[s],
                device_id=(right,),
                device_id_type=pl.DeviceIdType.MESH,
            )
            rdma.start()
            rdma.wait()

        r_copy.wait()
        y = (p_ref[osl, :] + rs_recv[N_DEV - 2].astype(jnp.float32)
             + r_chunk[...])
        ms = jnp.mean(y * y, axis=-1, keepdims=True)
        o = y * lax.rsqrt(ms + 1e-6) * g_ref[...]
        out_ref[osl, :] = o
        my_norm[...] = o.astype(jnp.bfloat16)

        for s in range(N_DEV - 1):
            src = my_norm if s == 0 else ag_recv.at[s - 1]
            rdma = pltpu.make_async_remote_copy(
                src_ref=src,
                dst_ref=ag_recv.at[s],
                send_sem=ag_send_sems.at[s],
                recv_sem=ag_recv_sems.at[s],
                device_id=(right,),
                device_id_type=pl.DeviceIdType.MESH,
            )
            rdma.start()
            rdma.wait()
            idx = lax.rem(my - s + N_DEV, N_DEV)
            out_ref[pl.ds(idx * CH, CH), :] = ag_recv[s].astype(jnp.float32)

    return pl.pallas_call(
        body,
        out_shape=jax.ShapeDtypeStruct((M, D), jnp.float32),
        in_specs=[
            pl.BlockSpec(memory_space=pltpu.VMEM),
            pl.BlockSpec(memory_space=pl.ANY),
            pl.BlockSpec(memory_space=pltpu.VMEM),
        ],
        out_specs=pl.BlockSpec(memory_space=pltpu.VMEM),
        scratch_shapes=[
            pltpu.VMEM((2, CH, D), jnp.bfloat16),
            pltpu.VMEM((N_DEV - 1, CH, D), jnp.bfloat16),
            pltpu.VMEM((CH, D), jnp.bfloat16),
            pltpu.VMEM((N_DEV - 1, CH, D), jnp.bfloat16),
            pltpu.VMEM((CH, D), jnp.float32),
            pltpu.SemaphoreType.DMA((N_DEV - 1,)),
            pltpu.SemaphoreType.DMA((N_DEV - 1,)),
            pltpu.SemaphoreType.DMA((N_DEV - 1,)),
            pltpu.SemaphoreType.DMA((N_DEV - 1,)),
            pltpu.SemaphoreType.DMA,
        ],
        compiler_params=pltpu.CompilerParams(
            vmem_limit_bytes=100 * 1024 * 1024,
        ),
    )(p2, resid, g2)
